# baseline (speedup 1.0000x reference)
import functools

import jax
import jax.numpy as jnp
from jax import lax
from jax.experimental import pallas as pl
from jax.experimental.pallas import tpu as pltpu
from jax.experimental.pallas import tpu_sc as plsc

N = 10000
E = 320000
DF = 128
DE = 16
DM = 256
HC = 128
HF = 128

NC = 2
NS = 16
NW = NC * NS
EPW = E // NW
EB = 40
NIT = EPW // EB
EBH = EB // 2
NV = NIT // 2
NPAD = 10240
RPT = NPAD // NS
ZB = EB
NZC = RPT // ZB

_LANES = 16
_ROW_CH = DF // _LANES
_WG = DF // 32
_PK = DF // 2


def _edge_agg_body(h_hbm, e_hbm, sd_hbm, out_hbm,
                   agg_sh, eb0, eb1, hb0, hb1, mg0, mg1, sd0, sd1, sd2, sd3,
                   es0, es1, gs0, gs1, cs0, cs1, ds0, ds1, ds2, ds3):
  c = lax.axis_index("c")
  s = lax.axis_index("s")
  wid = s * NC + c
  base = wid * EPW
  ebufs = (eb0, eb1)
  hbufs = (hb0, hb1)
  msgs = (mg0, mg1)
  sdbufs = (sd0, sd1, sd2, sd3)
  esems = (es0, es1)
  gsems = (gs0, gs1)
  csems = (cs0, cs1)
  dsems = (ds0, ds1, ds2, ds3)

  zeros16 = jnp.zeros((_LANES,), jnp.float32)

  @plsc.parallel_loop(0, ZB)
  def _(r):
    for j in range(_ROW_CH):
      mg0[r, pl.ds(j * _LANES, _LANES)] = zeros16

  for k in range(NZC):
    pltpu.sync_copy(mg0, agg_sh.at[pl.ds(s * RPT + k * ZB, ZB)])
  plsc.subcore_barrier()

  base2 = wid * (EPW // 2)

  def eload(v, u, wait=False):
    d = pltpu.make_async_copy(
        e_hbm.at[pl.ds(base2 + v * EB, EB)], ebufs[u], esems[u])
    d.wait() if wait else d.start()

  def sdload(k, q, wait=False):
    d = pltpu.make_async_copy(sd_hbm.at[wid, k], sdbufs[q], dsems[q])
    d.wait() if wait else d.start()

  def gather(q, b, wait=False):
    d = pltpu.make_async_copy(h_hbm.at[sdbufs[q].at[0]], hbufs[b], gsems[b])
    d.wait() if wait else d.start()

  def scatter(q, b, wait=False):
    if wait:
      pltpu.make_async_copy(msgs[b], agg_sh.at[sdbufs[q].at[1]],
                            csems[b]).wait()
    else:
      pltpu.async_copy(msgs[b], agg_sh.at[sdbufs[q].at[1]], csems[b],
                       add=True)

  himask = jnp.full((_LANES,), jnp.int32(-65536))

  def widen(w, high):
    if high:
      return lax.bitcast_convert_type(jnp.bitwise_and(w, himask),
                                      jnp.float32)
    return lax.bitcast_convert_type(lax.shift_left(w, 16), jnp.float32)

  def relu(b, u, half):
    eb = ebufs[u]
    hb = hbufs[b]
    mg = msgs[b]
    ebase = EBH * half

    @plsc.parallel_loop(0, EBH, unroll=2)
    def _(t):
      for g in range(_WG):
        we0 = eb[ebase + t, pl.ds(16 * g, _LANES)]
        we1 = eb[ebase + t, pl.ds(_PK + 16 * g, _LANES)]
        for i, we in ((2 * t, we0), (2 * t + 1, we1)):
          lo = hb[i, pl.ds(32 * g, _LANES)] + widen(we, False)
          hi = hb[i, pl.ds(32 * g + _LANES, _LANES)] + widen(we, True)
          mg[i, pl.ds(32 * g, _LANES)] = jnp.maximum(lo, 0.0)
          mg[i, pl.ds(32 * g + _LANES, _LANES)] = jnp.maximum(hi, 0.0)

  def block(k, v, q, b, u, half, first=False, next_gather=True, next_sd=True,
            eissue=True):
    if next_gather:
      sdload(k + 1, (q + 1) % 4, wait=True)
      gather((q + 1) % 4, b ^ 1)
    gather(q, b, wait=True)
    if half == 0:
      eload(v, u, wait=True)
    if not first:
      scatter((q + 2) % 4, b, wait=True)
    relu(b, u, half)
    scatter(q, b)
    if half == 1 and eissue:
      eload(v + 2, u)
    if next_sd:
      sdload(k + 2, (q + 2) % 4)

  sdload(0, 0)
  sdload(1, 1)
  eload(0, 0)
  eload(1, 1)
  sdload(0, 0, wait=True)
  gather(0, 0)
  block(0, 0, 0, 0, 0, 0, first=True)
  block(1, 0, 1, 1, 0, 1, first=True)

  def steady(sstep, carry):
    k = 4 * sstep - 2
    v = 2 * sstep - 1
    block(k, v, 2, 0, 1, 0)
    block(k + 1, v, 3, 1, 1, 1)
    block(k + 2, v + 1, 0, 0, 0, 0)
    block(k + 3, v + 1, 1, 1, 0, 1)
    return carry

  lax.fori_loop(1, (NIT - 2) // 4, steady, 0)
  block(NIT - 4, NV - 2, 2, 0, 1, 0)
  block(NIT - 3, NV - 2, 3, 1, 1, 1, eissue=False)
  block(NIT - 2, NV - 1, 0, 0, 0, 0, next_sd=False)
  block(NIT - 1, NV - 1, 1, 1, 0, 1,
        next_gather=False, next_sd=False, eissue=False)
  scatter(0, 0, wait=True)
  scatter(1, 1, wait=True)
  plsc.subcore_barrier()

  for k in range(NZC):
    row0 = s * RPT + k * ZB
    pltpu.sync_copy(agg_sh.at[pl.ds(row0, ZB)], mg0)
    pltpu.sync_copy(mg0, out_hbm.at[c, pl.ds(row0, ZB)])


@functools.cache
def _make_edge_agg():
  mesh = plsc.VectorSubcoreMesh(core_axis_name="c", subcore_axis_name="s",
                                num_cores=NC, num_subcores=NS)
  return pl.kernel(
      _edge_agg_body,
      out_type=jax.ShapeDtypeStruct((NC, NPAD, DF), jnp.float32),
      mesh=mesh,
      scratch_types=[
          pltpu.VMEM_SHARED((NPAD, DF), jnp.float32),
          pltpu.VMEM((EB, DF), jnp.int32),
          pltpu.VMEM((EB, DF), jnp.int32),
          pltpu.VMEM((EB, DF), jnp.float32),
          pltpu.VMEM((EB, DF), jnp.float32),
          pltpu.VMEM((EB, DF), jnp.float32),
          pltpu.VMEM((EB, DF), jnp.float32),
          pltpu.VMEM((2, EB), jnp.int32),
          pltpu.VMEM((2, EB), jnp.int32),
          pltpu.VMEM((2, EB), jnp.int32),
          pltpu.VMEM((2, EB), jnp.int32),
          pltpu.SemaphoreType.DMA,
          pltpu.SemaphoreType.DMA,
          pltpu.SemaphoreType.DMA,
          pltpu.SemaphoreType.DMA,
          pltpu.SemaphoreType.DMA,
          pltpu.SemaphoreType.DMA,
          pltpu.SemaphoreType.DMA,
          pltpu.SemaphoreType.DMA,
          pltpu.SemaphoreType.DMA,
          pltpu.SemaphoreType.DMA,
      ],
  )


def _edge_agg(h, e, src, dst):
  sd = jnp.concatenate(
      [src.reshape(NW, NIT, 1, EB), dst.reshape(NW, NIT, 1, EB)], axis=2)
  return _make_edge_agg()(h, e, sd)


def _pack_cols(z):
  outs = []
  for h2 in range(z.shape[1] // DF):
    for g in range(_WG):
      c0 = DF * h2 + 32 * g
      bl = lax.bitcast_convert_type(z[:, c0:c0 + 16], jnp.int32)
      bh = lax.bitcast_convert_type(z[:, c0 + 16:c0 + 32], jnp.int32)
      outs.append(jnp.bitwise_or(
          jnp.bitwise_and(bh + 32768, jnp.int32(-65536)),
          lax.shift_right_logical(bl + 32768, 16)))
  return jnp.concatenate(outs, axis=1)


_EMB = 2000
E2 = E // 2


def _edge_mm_body(ea_ref, w_ref, b_ref, o_ref):
  z = (jnp.dot(ea_ref[...], w_ref[...], preferred_element_type=jnp.float32)
       + b_ref[...])
  o_ref[...] = _pack_cols(z)


def _edge_mm(ea2, w2, b2):
  return pl.pallas_call(
      _edge_mm_body,
      grid=(E2 // _EMB,),
      in_specs=[
          pl.BlockSpec((_EMB, 2 * DE), lambda k: (k, 0)),
          pl.BlockSpec((2 * DE, 2 * DF), lambda k: (0, 0)),
          pl.BlockSpec((1, 2 * DF), lambda k: (0, 0)),
      ],
      out_specs=pl.BlockSpec((_EMB, DF), lambda k: (k, 0)),
      out_shape=jax.ShapeDtypeStruct((E2, DF), jnp.int32),
  )(ea2, w2, b2)


def _pack_x_body(x_ref, o_ref):
  o_ref[...] = _pack_cols(x_ref[...])


def _pack_x(x):
  return pl.pallas_call(
      _pack_x_body,
      grid=(5,),
      in_specs=[pl.BlockSpec((N // 5, DF), lambda k: (k, 0))],
      out_specs=pl.BlockSpec((N // 5, _PK), lambda k: (k, 0)),
      out_shape=jax.ShapeDtypeStruct((N, _PK), jnp.int32),
  )(x)


_NRB = 2000
_NG = N // _NRB


def _stats_update(k, z, s_ref, ss_ref):
  cs = jnp.sum(z, axis=0, keepdims=True)
  css = jnp.sum(z * z, axis=0, keepdims=True)

  @pl.when(k == 0)
  def _():
    s_ref[...] = cs
    ss_ref[...] = css

  @pl.when(k != 0)
  def _():
    s_ref[...] += cs
    ss_ref[...] += css


def _conv_mm1_body(h_ref, eps_ref, a0_ref, a1_ref, w_ref, b_ref,
                   z_ref, s_ref, ss_ref):
  k = pl.program_id(0)
  y = h_ref[...] * eps_ref[...] + a0_ref[...] + a1_ref[...]
  z = jnp.dot(y, w_ref[...], preferred_element_type=jnp.float32) + b_ref[...]
  z_ref[...] = z
  _stats_update(k, z, s_ref, ss_ref)


def _conv_mm1(h, eps, a0, a1, w, b):
  return pl.pallas_call(
      _conv_mm1_body,
      grid=(_NG,),
      in_specs=[
          pl.BlockSpec((_NRB, DF), lambda k: (k, 0)),
          pl.BlockSpec((1, DF), lambda k: (0, 0)),
          pl.BlockSpec((_NRB, DF), lambda k: (k, 0)),
          pl.BlockSpec((_NRB, DF), lambda k: (k, 0)),
          pl.BlockSpec((DF, HC), lambda k: (0, 0)),
          pl.BlockSpec((1, HC), lambda k: (0, 0)),
      ],
      out_specs=[
          pl.BlockSpec((_NRB, HC), lambda k: (k, 0)),
          pl.BlockSpec((1, HC), lambda k: (0, 0)),
          pl.BlockSpec((1, HC), lambda k: (0, 0)),
      ],
      out_shape=[
          jax.ShapeDtypeStruct((N, HC), jnp.float32),
          jax.ShapeDtypeStruct((1, HC), jnp.float32),
          jax.ShapeDtypeStruct((1, HC), jnp.float32),
      ],
  )(h, eps, a0, a1, w, b)


def _bn_cols(z, s, ss, g, c):
  m = s * (1.0 / N)
  v = ss * (1.0 / N) - m * m
  inv = g * lax.rsqrt(v + 1e-5)
  return (z - m) * inv + c


def _leaky(x):
  return jnp.where(x >= 0, x, 0.01 * x)


def _bn_mm2_body(z_ref, s_ref, ss_ref, g_ref, c_ref, w_ref, b_ref,
                 o_ref, s2_ref, ss2_ref):
  k = pl.program_id(0)
  t = _leaky(_bn_cols(z_ref[...], s_ref[...], ss_ref[...],
                      g_ref[...], c_ref[...]))
  z2 = jnp.dot(t, w_ref[...], preferred_element_type=jnp.float32) + b_ref[...]
  o_ref[...] = z2
  _stats_update(k, z2, s2_ref, ss2_ref)


def _bn_mm2(z, s, ss, g, c, w, b):
  dout = w.shape[1]
  return pl.pallas_call(
      _bn_mm2_body,
      grid=(_NG,),
      in_specs=[
          pl.BlockSpec((_NRB, HC), lambda k: (k, 0)),
          pl.BlockSpec((1, HC), lambda k: (0, 0)),
          pl.BlockSpec((1, HC), lambda k: (0, 0)),
          pl.BlockSpec((1, HC), lambda k: (0, 0)),
          pl.BlockSpec((1, HC), lambda k: (0, 0)),
          pl.BlockSpec((HC, dout), lambda k: (0, 0)),
          pl.BlockSpec((1, dout), lambda k: (0, 0)),
      ],
      out_specs=[
          pl.BlockSpec((_NRB, dout), lambda k: (k, 0)),
          pl.BlockSpec((1, dout), lambda k: (0, 0)),
          pl.BlockSpec((1, dout), lambda k: (0, 0)),
      ],
      out_shape=[
          jax.ShapeDtypeStruct((N, dout), jnp.float32),
          jax.ShapeDtypeStruct((1, dout), jnp.float32),
          jax.ShapeDtypeStruct((1, dout), jnp.float32),
      ],
  )(z, s, ss, g, c, w, b)


def _bn_leaky_body(z_ref, s_ref, ss_ref, g_ref, c_ref, o_ref):
  o_ref[...] = _leaky(_bn_cols(z_ref[...], s_ref[...], ss_ref[...],
                               g_ref[...], c_ref[...]))


def _bn_leaky(z, s, ss, g, c):
  return pl.pallas_call(
      _bn_leaky_body,
      grid=(_NG,),
      in_specs=[
          pl.BlockSpec((_NRB, HC), lambda k: (k, 0)),
          pl.BlockSpec((1, HC), lambda k: (0, 0)),
          pl.BlockSpec((1, HC), lambda k: (0, 0)),
          pl.BlockSpec((1, HC), lambda k: (0, 0)),
          pl.BlockSpec((1, HC), lambda k: (0, 0)),
      ],
      out_specs=pl.BlockSpec((_NRB, HC), lambda k: (k, 0)),
      out_shape=jax.ShapeDtypeStruct((N, HC), jnp.float32),
  )(z, s, ss, g, c)


def _final_mm1_body(h_ref, mx_ref, wa_ref, wb_ref, b_ref, z_ref, s_ref, ss_ref):
  k = pl.program_id(0)
  z = (jnp.dot(h_ref[...], wa_ref[...], preferred_element_type=jnp.float32)
       + jnp.dot(mx_ref[...], wb_ref[...], preferred_element_type=jnp.float32)
       + b_ref[...])
  z_ref[...] = z
  _stats_update(k, z, s_ref, ss_ref)


def _final_mm1(h, mx, wa, wb, b):
  return pl.pallas_call(
      _final_mm1_body,
      grid=(_NG,),
      in_specs=[
          pl.BlockSpec((_NRB, HC), lambda k: (k, 0)),
          pl.BlockSpec((_NRB, DM), lambda k: (k, 0)),
          pl.BlockSpec((HC, HF), lambda k: (0, 0)),
          pl.BlockSpec((DM, HF), lambda k: (0, 0)),
          pl.BlockSpec((1, HF), lambda k: (0, 0)),
      ],
      out_specs=[
          pl.BlockSpec((_NRB, HF), lambda k: (k, 0)),
          pl.BlockSpec((1, HF), lambda k: (0, 0)),
          pl.BlockSpec((1, HF), lambda k: (0, 0)),
      ],
      out_shape=[
          jax.ShapeDtypeStruct((N, HF), jnp.float32),
          jax.ShapeDtypeStruct((1, HF), jnp.float32),
          jax.ShapeDtypeStruct((1, HF), jnp.float32),
      ],
  )(h, mx, wa, wb, b)


def kernel(x, edge_index, edge_attr, mol_x, params):
  src = edge_index[0]
  dst = edge_index[1]

  def row(v):
    return v.reshape(1, -1)

  ea2 = edge_attr.reshape(E2, 2 * DE)
  es = []
  for i in range(3):
    p = params["conv%d" % i]
    w2 = jnp.zeros((2 * DE, 2 * DF), jnp.float32)
    w2 = w2.at[:DE, :DF].set(p["We"]).at[DE:, DF:].set(p["We"])
    b2 = jnp.concatenate([p["be"], p["be"]]).reshape(1, 2 * DF)
    es.append(_edge_mm(ea2, w2, b2))

  h = x
  for i in range(3):
    p = params["conv%d" % i]
    parts = _edge_agg(h, es[i], src, dst)
    epsb = jnp.broadcast_to(1.0 + p["eps"], (1, DF)).astype(jnp.float32)
    z1, s1, ss1 = _conv_mm1(h, epsb, parts[0], parts[1],
                            p["W1"], row(p["b1"]))
    z2, s2, ss2 = _bn_mm2(z1, s1, ss1, row(p["g1"]), row(p["c1"]),
                          p["W2"], row(p["b2"]))
    if i != 2:
      h = _bn_leaky(z2, s2, ss2, row(p["go"]), row(p["co"]))
    else:
      h = z2

  pf = params["final"]
  wa = pf["W1"][:HC]
  wb = pf["W1"][HC:]
  o1, fs, fss = _final_mm1(h, mol_x, wa, wb, row(pf["b1"]))
  w2p = jnp.zeros((HF, 128), jnp.float32).at[:, :1].set(pf["W2"])
  b2p = jnp.zeros((1, 128), jnp.float32).at[0, 0].set(pf["b2"][0])
  o, _, _ = _bn_mm2(o1, fs, fss, row(pf["g"]), row(pf["c"]), w2p, b2p)
  return o[:, 0]


# speedup vs baseline: 1.1610x; 1.1610x over previous
import functools

import jax
import jax.numpy as jnp
from jax import lax
from jax.experimental import pallas as pl
from jax.experimental.pallas import tpu as pltpu
from jax.experimental.pallas import tpu_sc as plsc

N = 10000
E = 320000
DF = 128
DE = 16
DM = 256
HC = 128
HF = 128

NC = 2
NS = 16
NW = NC * NS
EPW = E // NW
EB = 80
NIT = EPW // EB
EBH = EB // 2
NPAD = 10240
RPT = NPAD // NS
ZB = EB
NZC = RPT // ZB

_LANES = 16
_ROW_CH = DF // _LANES
_WG = DF // 32
_PK = DF // 2


def _edge_agg_body(h_hbm, e_hbm, sd_hbm, out_hbm,
                   agg_sh, eb0, eb1, hm0, hm1, hm2, sd0, sd1, sd2, sd3,
                   es0, es1, gs0, gs1, gs2, cs0, cs1, cs2, ds0, ds1, ds2,
                   ds3):
  c = lax.axis_index("c")
  s = lax.axis_index("s")
  wid = s * NC + c
  ebufs = (eb0, eb1)
  hms = (hm0, hm1, hm2)
  sdbufs = (sd0, sd1, sd2, sd3)
  esems = (es0, es1)
  gsems = (gs0, gs1, gs2)
  csems = (cs0, cs1, cs2)
  dsems = (ds0, ds1, ds2, ds3)

  zeros16 = jnp.zeros((_LANES,), jnp.float32)

  @plsc.parallel_loop(0, ZB)
  def _(r):
    for j in range(_ROW_CH):
      hm0[r, pl.ds(j * _LANES, _LANES)] = zeros16

  for k in range(NZC):
    pltpu.sync_copy(hm0, agg_sh.at[pl.ds(s * RPT + k * ZB, ZB)])
  plsc.subcore_barrier()

  base2 = wid * (EPW // 2)

  def eload(k, u, wait=False):
    d = pltpu.make_async_copy(
        e_hbm.at[pl.ds(base2 + k * EBH, EBH)], ebufs[u], esems[u])
    d.wait() if wait else d.start()

  def sdload(k, q, wait=False):
    d = pltpu.make_async_copy(sd_hbm.at[wid, k], sdbufs[q], dsems[q])
    d.wait() if wait else d.start()

  def gather(q, j, wait=False):
    d = pltpu.make_async_copy(h_hbm.at[sdbufs[q].at[0]], hms[j], gsems[j])
    d.wait() if wait else d.start()

  def scatter(q, j, wait=False):
    if wait:
      pltpu.make_async_copy(hms[j], agg_sh.at[sdbufs[q].at[1]],
                            csems[j]).wait()
    else:
      pltpu.async_copy(hms[j], agg_sh.at[sdbufs[q].at[1]], csems[j],
                       add=True)

  himask = jnp.full((_LANES,), jnp.int32(-65536))

  def widen(w, high):
    if high:
      return lax.bitcast_convert_type(jnp.bitwise_and(w, himask),
                                      jnp.float32)
    return lax.bitcast_convert_type(lax.shift_left(w, 16), jnp.float32)

  def relu(j, u):
    eb = ebufs[u]
    hm = hms[j]

    @plsc.parallel_loop(0, EBH, unroll=2)
    def _(t):
      for g in range(_WG):
        we0 = eb[t, pl.ds(16 * g, _LANES)]
        we1 = eb[t, pl.ds(_PK + 16 * g, _LANES)]
        for i, we in ((2 * t, we0), (2 * t + 1, we1)):
          lo = hm[i, pl.ds(32 * g, _LANES)] + widen(we, False)
          hi = hm[i, pl.ds(32 * g + _LANES, _LANES)] + widen(we, True)
          hm[i, pl.ds(32 * g, _LANES)] = jnp.maximum(lo, 0.0)
          hm[i, pl.ds(32 * g + _LANES, _LANES)] = jnp.maximum(hi, 0.0)

  def block(k, q, j, u, first=False, next_gather=True, next_sd=True,
            eissue=True):
    if not first:
      scatter((q + 2) % 4, (j + 1) % 3, wait=True)
    if next_gather:
      sdload(k + 1, (q + 1) % 4, wait=True)
      gather((q + 1) % 4, (j + 1) % 3)
    gather(q, j, wait=True)
    eload(k, u, wait=True)
    relu(j, u)
    scatter(q, j)
    if eissue:
      eload(k + 2, u)
    if next_sd:
      sdload(k + 2, (q + 2) % 4)

  sdload(0, 0)
  sdload(1, 1)
  eload(0, 0)
  eload(1, 1)
  sdload(0, 0, wait=True)
  gather(0, 0)
  block(0, 0, 0, 0, first=True)
  block(1, 1, 1, 1, first=True)

  def steady(sstep, carry):
    k0 = 2 + 12 * sstep
    for p in range(12):
      block(k0 + p, (2 + p) % 4, (2 + p) % 3, p % 2)
    return carry

  lax.fori_loop(0, (NIT - 5) // 12, steady, 0)
  block(NIT - 3, (NIT - 3) % 4, (NIT - 3) % 3, (NIT - 3) % 2)
  block(NIT - 2, (NIT - 2) % 4, (NIT - 2) % 3, (NIT - 2) % 2, eissue=False,
        next_sd=False)
  block(NIT - 1, (NIT - 1) % 4, (NIT - 1) % 3, (NIT - 1) % 2,
        next_gather=False, next_sd=False, eissue=False)
  scatter((NIT - 2) % 4, (NIT - 2) % 3, wait=True)
  scatter((NIT - 1) % 4, (NIT - 1) % 3, wait=True)
  plsc.subcore_barrier()

  for k in range(NZC):
    row0 = s * RPT + k * ZB
    pltpu.sync_copy(agg_sh.at[pl.ds(row0, ZB)], hm0)
    pltpu.sync_copy(hm0, out_hbm.at[c, pl.ds(row0, ZB)])


@functools.cache
def _make_edge_agg():
  mesh = plsc.VectorSubcoreMesh(core_axis_name="c", subcore_axis_name="s",
                                num_cores=NC, num_subcores=NS)
  return pl.kernel(
      _edge_agg_body,
      out_type=jax.ShapeDtypeStruct((NC, NPAD, DF), jnp.float32),
      mesh=mesh,
      scratch_types=[
          pltpu.VMEM_SHARED((NPAD, DF), jnp.float32),
          pltpu.VMEM((EBH, DF), jnp.int32),
          pltpu.VMEM((EBH, DF), jnp.int32),
          pltpu.VMEM((EB, DF), jnp.float32),
          pltpu.VMEM((EB, DF), jnp.float32),
          pltpu.VMEM((EB, DF), jnp.float32),
          pltpu.VMEM((2, EB), jnp.int32),
          pltpu.VMEM((2, EB), jnp.int32),
          pltpu.VMEM((2, EB), jnp.int32),
          pltpu.VMEM((2, EB), jnp.int32),
          pltpu.SemaphoreType.DMA,
          pltpu.SemaphoreType.DMA,
          pltpu.SemaphoreType.DMA,
          pltpu.SemaphoreType.DMA,
          pltpu.SemaphoreType.DMA,
          pltpu.SemaphoreType.DMA,
          pltpu.SemaphoreType.DMA,
          pltpu.SemaphoreType.DMA,
          pltpu.SemaphoreType.DMA,
          pltpu.SemaphoreType.DMA,
          pltpu.SemaphoreType.DMA,
          pltpu.SemaphoreType.DMA,
      ],
  )


def _edge_agg(h, e, src, dst):
  sd = jnp.concatenate(
      [src.reshape(NW, NIT, 1, EB), dst.reshape(NW, NIT, 1, EB)], axis=2)
  return _make_edge_agg()(h, e, sd)


def _pack_cols(z):
  outs = []
  for h2 in range(z.shape[1] // DF):
    for g in range(_WG):
      c0 = DF * h2 + 32 * g
      bl = lax.bitcast_convert_type(z[:, c0:c0 + 16], jnp.int32)
      bh = lax.bitcast_convert_type(z[:, c0 + 16:c0 + 32], jnp.int32)
      outs.append(jnp.bitwise_or(
          jnp.bitwise_and(bh + 32768, jnp.int32(-65536)),
          lax.shift_right_logical(bl + 32768, 16)))
  return jnp.concatenate(outs, axis=1)


_EMB = 2000
E2 = E // 2


def _edge_mm_body(ea_ref, w_ref, b_ref, o_ref):
  z = (jnp.dot(ea_ref[...], w_ref[...], preferred_element_type=jnp.float32)
       + b_ref[...])
  o_ref[...] = _pack_cols(z)


def _edge_mm(ea2, w2, b2):
  return pl.pallas_call(
      _edge_mm_body,
      grid=(E2 // _EMB,),
      in_specs=[
          pl.BlockSpec((_EMB, 2 * DE), lambda k: (k, 0)),
          pl.BlockSpec((2 * DE, 2 * DF), lambda k: (0, 0)),
          pl.BlockSpec((1, 2 * DF), lambda k: (0, 0)),
      ],
      out_specs=pl.BlockSpec((_EMB, DF), lambda k: (k, 0)),
      out_shape=jax.ShapeDtypeStruct((E2, DF), jnp.int32),
  )(ea2, w2, b2)


def _pack_x_body(x_ref, o_ref):
  o_ref[...] = _pack_cols(x_ref[...])


def _pack_x(x):
  return pl.pallas_call(
      _pack_x_body,
      grid=(5,),
      in_specs=[pl.BlockSpec((N // 5, DF), lambda k: (k, 0))],
      out_specs=pl.BlockSpec((N // 5, _PK), lambda k: (k, 0)),
      out_shape=jax.ShapeDtypeStruct((N, _PK), jnp.int32),
  )(x)


_NRB = 2000
_NG = N // _NRB


def _stats_update(k, z, s_ref, ss_ref):
  cs = jnp.sum(z, axis=0, keepdims=True)
  css = jnp.sum(z * z, axis=0, keepdims=True)

  @pl.when(k == 0)
  def _():
    s_ref[...] = cs
    ss_ref[...] = css

  @pl.when(k != 0)
  def _():
    s_ref[...] += cs
    ss_ref[...] += css


def _conv_mm1_body(h_ref, eps_ref, a0_ref, a1_ref, w_ref, b_ref,
                   z_ref, s_ref, ss_ref):
  k = pl.program_id(0)
  y = h_ref[...] * eps_ref[...] + a0_ref[...] + a1_ref[...]
  z = jnp.dot(y, w_ref[...], preferred_element_type=jnp.float32) + b_ref[...]
  z_ref[...] = z
  _stats_update(k, z, s_ref, ss_ref)


def _conv_mm1(h, eps, a0, a1, w, b):
  return pl.pallas_call(
      _conv_mm1_body,
      grid=(_NG,),
      in_specs=[
          pl.BlockSpec((_NRB, DF), lambda k: (k, 0)),
          pl.BlockSpec((1, DF), lambda k: (0, 0)),
          pl.BlockSpec((_NRB, DF), lambda k: (k, 0)),
          pl.BlockSpec((_NRB, DF), lambda k: (k, 0)),
          pl.BlockSpec((DF, HC), lambda k: (0, 0)),
          pl.BlockSpec((1, HC), lambda k: (0, 0)),
      ],
      out_specs=[
          pl.BlockSpec((_NRB, HC), lambda k: (k, 0)),
          pl.BlockSpec((1, HC), lambda k: (0, 0)),
          pl.BlockSpec((1, HC), lambda k: (0, 0)),
      ],
      out_shape=[
          jax.ShapeDtypeStruct((N, HC), jnp.float32),
          jax.ShapeDtypeStruct((1, HC), jnp.float32),
          jax.ShapeDtypeStruct((1, HC), jnp.float32),
      ],
  )(h, eps, a0, a1, w, b)


def _bn_cols(z, s, ss, g, c):
  m = s * (1.0 / N)
  v = ss * (1.0 / N) - m * m
  inv = g * lax.rsqrt(v + 1e-5)
  return (z - m) * inv + c


def _leaky(x):
  return jnp.where(x >= 0, x, 0.01 * x)


def _bn_mm2_body(z_ref, s_ref, ss_ref, g_ref, c_ref, w_ref, b_ref,
                 o_ref, s2_ref, ss2_ref):
  k = pl.program_id(0)
  t = _leaky(_bn_cols(z_ref[...], s_ref[...], ss_ref[...],
                      g_ref[...], c_ref[...]))
  z2 = jnp.dot(t, w_ref[...], preferred_element_type=jnp.float32) + b_ref[...]
  o_ref[...] = z2
  _stats_update(k, z2, s2_ref, ss2_ref)


def _bn_mm2(z, s, ss, g, c, w, b):
  dout = w.shape[1]
  return pl.pallas_call(
      _bn_mm2_body,
      grid=(_NG,),
      in_specs=[
          pl.BlockSpec((_NRB, HC), lambda k: (k, 0)),
          pl.BlockSpec((1, HC), lambda k: (0, 0)),
          pl.BlockSpec((1, HC), lambda k: (0, 0)),
          pl.BlockSpec((1, HC), lambda k: (0, 0)),
          pl.BlockSpec((1, HC), lambda k: (0, 0)),
          pl.BlockSpec((HC, dout), lambda k: (0, 0)),
          pl.BlockSpec((1, dout), lambda k: (0, 0)),
      ],
      out_specs=[
          pl.BlockSpec((_NRB, dout), lambda k: (k, 0)),
          pl.BlockSpec((1, dout), lambda k: (0, 0)),
          pl.BlockSpec((1, dout), lambda k: (0, 0)),
      ],
      out_shape=[
          jax.ShapeDtypeStruct((N, dout), jnp.float32),
          jax.ShapeDtypeStruct((1, dout), jnp.float32),
          jax.ShapeDtypeStruct((1, dout), jnp.float32),
      ],
  )(z, s, ss, g, c, w, b)


def _bn_leaky_body(z_ref, s_ref, ss_ref, g_ref, c_ref, o_ref):
  o_ref[...] = _leaky(_bn_cols(z_ref[...], s_ref[...], ss_ref[...],
                               g_ref[...], c_ref[...]))


def _bn_leaky(z, s, ss, g, c):
  return pl.pallas_call(
      _bn_leaky_body,
      grid=(_NG,),
      in_specs=[
          pl.BlockSpec((_NRB, HC), lambda k: (k, 0)),
          pl.BlockSpec((1, HC), lambda k: (0, 0)),
          pl.BlockSpec((1, HC), lambda k: (0, 0)),
          pl.BlockSpec((1, HC), lambda k: (0, 0)),
          pl.BlockSpec((1, HC), lambda k: (0, 0)),
      ],
      out_specs=pl.BlockSpec((_NRB, HC), lambda k: (k, 0)),
      out_shape=jax.ShapeDtypeStruct((N, HC), jnp.float32),
  )(z, s, ss, g, c)


def _final_mm1_body(h_ref, mx_ref, wa_ref, wb_ref, b_ref, z_ref, s_ref, ss_ref):
  k = pl.program_id(0)
  z = (jnp.dot(h_ref[...], wa_ref[...], preferred_element_type=jnp.float32)
       + jnp.dot(mx_ref[...], wb_ref[...], preferred_element_type=jnp.float32)
       + b_ref[...])
  z_ref[...] = z
  _stats_update(k, z, s_ref, ss_ref)


def _final_mm1(h, mx, wa, wb, b):
  return pl.pallas_call(
      _final_mm1_body,
      grid=(_NG,),
      in_specs=[
          pl.BlockSpec((_NRB, HC), lambda k: (k, 0)),
          pl.BlockSpec((_NRB, DM), lambda k: (k, 0)),
          pl.BlockSpec((HC, HF), lambda k: (0, 0)),
          pl.BlockSpec((DM, HF), lambda k: (0, 0)),
          pl.BlockSpec((1, HF), lambda k: (0, 0)),
      ],
      out_specs=[
          pl.BlockSpec((_NRB, HF), lambda k: (k, 0)),
          pl.BlockSpec((1, HF), lambda k: (0, 0)),
          pl.BlockSpec((1, HF), lambda k: (0, 0)),
      ],
      out_shape=[
          jax.ShapeDtypeStruct((N, HF), jnp.float32),
          jax.ShapeDtypeStruct((1, HF), jnp.float32),
          jax.ShapeDtypeStruct((1, HF), jnp.float32),
      ],
  )(h, mx, wa, wb, b)


def kernel(x, edge_index, edge_attr, mol_x, params):
  src = edge_index[0]
  dst = edge_index[1]

  def row(v):
    return v.reshape(1, -1)

  ea2 = edge_attr.reshape(E2, 2 * DE)
  es = []
  for i in range(3):
    p = params["conv%d" % i]
    w2 = jnp.zeros((2 * DE, 2 * DF), jnp.float32)
    w2 = w2.at[:DE, :DF].set(p["We"]).at[DE:, DF:].set(p["We"])
    b2 = jnp.concatenate([p["be"], p["be"]]).reshape(1, 2 * DF)
    es.append(_edge_mm(ea2, w2, b2))

  h = x
  for i in range(3):
    p = params["conv%d" % i]
    parts = _edge_agg(h, es[i], src, dst)
    epsb = jnp.broadcast_to(1.0 + p["eps"], (1, DF)).astype(jnp.float32)
    z1, s1, ss1 = _conv_mm1(h, epsb, parts[0], parts[1],
                            p["W1"], row(p["b1"]))
    z2, s2, ss2 = _bn_mm2(z1, s1, ss1, row(p["g1"]), row(p["c1"]),
                          p["W2"], row(p["b2"]))
    if i != 2:
      h = _bn_leaky(z2, s2, ss2, row(p["go"]), row(p["co"]))
    else:
      h = z2

  pf = params["final"]
  wa = pf["W1"][:HC]
  wb = pf["W1"][HC:]
  o1, fs, fss = _final_mm1(h, mol_x, wa, wb, row(pf["b1"]))
  w2p = jnp.zeros((HF, 128), jnp.float32).at[:, :1].set(pf["W2"])
  b2p = jnp.zeros((1, 128), jnp.float32).at[0, 0].set(pf["b2"][0])
  o, _, _ = _bn_mm2(o1, fs, fss, row(pf["g"]), row(pf["c"]), w2p, b2p)
  return o[:, 0]


# speedup vs baseline: 1.1628x; 1.0016x over previous
import functools

import jax
import jax.numpy as jnp
from jax import lax
from jax.experimental import pallas as pl
from jax.experimental.pallas import tpu as pltpu
from jax.experimental.pallas import tpu_sc as plsc

N = 10000
E = 320000
DF = 128
DE = 16
DM = 256
HC = 128
HF = 128

NC = 2
NS = 16
NW = NC * NS
EPW = E // NW
EB = 80
NIT = EPW // EB
EBH = EB // 2
NPAD = 10240
RPT = NPAD // NS
ZB = EB
NZC = RPT // ZB

_LANES = 16
_ROW_CH = DF // _LANES
_WG = DF // 32
_PK = DF // 2


def _edge_agg_body(h_hbm, e_hbm, sd_hbm, out_hbm,
                   agg_sh, eb0, eb1, hm0, hm1, hm2, sd0, sd1, sd2, sd3,
                   es0, es1, gs0, gs1, gs2, cs0, cs1, cs2, ds0, ds1, ds2,
                   ds3):
  c = lax.axis_index("c")
  s = lax.axis_index("s")
  wid = s * NC + c
  ebufs = (eb0, eb1)
  hms = (hm0, hm1, hm2)
  sdbufs = (sd0, sd1, sd2, sd3)
  esems = (es0, es1)
  gsems = (gs0, gs1, gs2)
  csems = (cs0, cs1, cs2)
  dsems = (ds0, ds1, ds2, ds3)

  zeros16 = jnp.zeros((_LANES,), jnp.float32)

  @plsc.parallel_loop(0, ZB)
  def _(r):
    for j in range(_ROW_CH):
      hm0[r, pl.ds(j * _LANES, _LANES)] = zeros16

  for k in range(NZC):
    pltpu.sync_copy(hm0, agg_sh.at[pl.ds(s * RPT + k * ZB, ZB)])
  plsc.subcore_barrier()

  base2 = wid * (EPW // 2)

  def eload(k, u, wait=False):
    d = pltpu.make_async_copy(
        e_hbm.at[pl.ds(base2 + k * EBH, EBH)], ebufs[u], esems[u])
    d.wait() if wait else d.start()

  def sdload(k, q, wait=False):
    d = pltpu.make_async_copy(sd_hbm.at[wid, k], sdbufs[q], dsems[q])
    d.wait() if wait else d.start()

  def gather(q, j, wait=False):
    d = pltpu.make_async_copy(h_hbm.at[sdbufs[q].at[0]], hms[j], gsems[j])
    d.wait() if wait else d.start()

  def scatter(q, j, wait=False):
    if wait:
      pltpu.make_async_copy(hms[j], agg_sh.at[sdbufs[q].at[1]],
                            csems[j]).wait()
    else:
      pltpu.async_copy(hms[j], agg_sh.at[sdbufs[q].at[1]], csems[j],
                       add=True)

  himask = jnp.full((_LANES,), jnp.int32(-65536))

  def widen(w, high):
    if high:
      return lax.bitcast_convert_type(jnp.bitwise_and(w, himask),
                                      jnp.float32)
    return lax.bitcast_convert_type(lax.shift_left(w, 16), jnp.float32)

  def relu(j, u):
    eb = ebufs[u]
    hm = hms[j]

    @plsc.parallel_loop(0, EBH, unroll=2)
    def _(t):
      for g in range(_WG):
        we0 = eb[t, pl.ds(16 * g, _LANES)]
        we1 = eb[t, pl.ds(_PK + 16 * g, _LANES)]
        for i, we in ((2 * t, we0), (2 * t + 1, we1)):
          lo = hm[i, pl.ds(32 * g, _LANES)] + widen(we, False)
          hi = hm[i, pl.ds(32 * g + _LANES, _LANES)] + widen(we, True)
          hm[i, pl.ds(32 * g, _LANES)] = jnp.maximum(lo, 0.0)
          hm[i, pl.ds(32 * g + _LANES, _LANES)] = jnp.maximum(hi, 0.0)

  def block(k, q, j, u, first=False, next_gather=True, next_sd=True,
            eissue=True):
    if not first:
      scatter((q + 2) % 4, (j + 1) % 3, wait=True)
    if next_gather:
      sdload(k + 1, (q + 1) % 4, wait=True)
      gather((q + 1) % 4, (j + 1) % 3)
    gather(q, j, wait=True)
    eload(k, u, wait=True)
    relu(j, u)
    scatter(q, j)
    if eissue:
      eload(k + 2, u)
    if next_sd:
      sdload(k + 2, (q + 2) % 4)

  sdload(0, 0)
  sdload(1, 1)
  eload(0, 0)
  eload(1, 1)
  sdload(0, 0, wait=True)
  gather(0, 0)
  block(0, 0, 0, 0, first=True)
  block(1, 1, 1, 1, first=True)

  def steady(sstep, carry):
    k0 = 2 + 12 * sstep
    for p in range(12):
      block(k0 + p, (2 + p) % 4, (2 + p) % 3, p % 2)
    return carry

  lax.fori_loop(0, (NIT - 5) // 12, steady, 0)
  block(NIT - 3, (NIT - 3) % 4, (NIT - 3) % 3, (NIT - 3) % 2)
  block(NIT - 2, (NIT - 2) % 4, (NIT - 2) % 3, (NIT - 2) % 2, eissue=False,
        next_sd=False)
  block(NIT - 1, (NIT - 1) % 4, (NIT - 1) % 3, (NIT - 1) % 2,
        next_gather=False, next_sd=False, eissue=False)
  scatter((NIT - 2) % 4, (NIT - 2) % 3, wait=True)
  scatter((NIT - 1) % 4, (NIT - 1) % 3, wait=True)
  plsc.subcore_barrier()

  for k in range(NZC):
    row0 = s * RPT + k * ZB
    pltpu.sync_copy(agg_sh.at[pl.ds(row0, ZB)], hm0)
    pltpu.sync_copy(hm0, out_hbm.at[c, pl.ds(row0, ZB)])


@functools.cache
def _make_edge_agg():
  mesh = plsc.VectorSubcoreMesh(core_axis_name="c", subcore_axis_name="s",
                                num_cores=NC, num_subcores=NS)
  return pl.kernel(
      _edge_agg_body,
      out_type=jax.ShapeDtypeStruct((NC, NPAD, DF), jnp.float32),
      mesh=mesh,
      scratch_types=[
          pltpu.VMEM_SHARED((NPAD, DF), jnp.float32),
          pltpu.VMEM((EBH, DF), jnp.int32),
          pltpu.VMEM((EBH, DF), jnp.int32),
          pltpu.VMEM((EB, DF), jnp.float32),
          pltpu.VMEM((EB, DF), jnp.float32),
          pltpu.VMEM((EB, DF), jnp.float32),
          pltpu.VMEM((2, EB), jnp.int32),
          pltpu.VMEM((2, EB), jnp.int32),
          pltpu.VMEM((2, EB), jnp.int32),
          pltpu.VMEM((2, EB), jnp.int32),
          pltpu.SemaphoreType.DMA,
          pltpu.SemaphoreType.DMA,
          pltpu.SemaphoreType.DMA,
          pltpu.SemaphoreType.DMA,
          pltpu.SemaphoreType.DMA,
          pltpu.SemaphoreType.DMA,
          pltpu.SemaphoreType.DMA,
          pltpu.SemaphoreType.DMA,
          pltpu.SemaphoreType.DMA,
          pltpu.SemaphoreType.DMA,
          pltpu.SemaphoreType.DMA,
          pltpu.SemaphoreType.DMA,
      ],
  )


def _edge_agg(h, e, src, dst):
  sd = jnp.concatenate(
      [src.reshape(NW, NIT, 1, EB), dst.reshape(NW, NIT, 1, EB)], axis=2)
  return _make_edge_agg()(h, e, sd)


def _pack_cols(z):
  outs = []
  for h2 in range(z.shape[1] // DF):
    for g in range(_WG):
      c0 = DF * h2 + 32 * g
      bl = lax.bitcast_convert_type(z[:, c0:c0 + 16], jnp.int32)
      bh = lax.bitcast_convert_type(z[:, c0 + 16:c0 + 32], jnp.int32)
      outs.append(jnp.bitwise_or(
          jnp.bitwise_and(bh + 32768, jnp.int32(-65536)),
          lax.shift_right_logical(bl + 32768, 16)))
  return jnp.concatenate(outs, axis=1)


_EMB = 2000
E2 = E // 2


def _edge_mm_body(ea_ref, w_ref, b_ref, o_ref):
  z = (jnp.dot(ea_ref[...], w_ref[...], preferred_element_type=jnp.float32)
       + b_ref[...])
  o_ref[...] = _pack_cols(z)


def _edge_mm(ea2, w2, b2):
  return pl.pallas_call(
      _edge_mm_body,
      grid=(E2 // _EMB,),
      in_specs=[
          pl.BlockSpec((_EMB, 2 * DE), lambda k: (k, 0)),
          pl.BlockSpec((2 * DE, 2 * DF), lambda k: (0, 0)),
          pl.BlockSpec((1, 2 * DF), lambda k: (0, 0)),
      ],
      out_specs=pl.BlockSpec((_EMB, DF), lambda k: (k, 0)),
      out_shape=jax.ShapeDtypeStruct((E2, DF), jnp.int32),
  )(ea2, w2, b2)


_NRB = 2000
_NG = N // _NRB


def _stats_update(k, z, s_ref, ss_ref):
  cs = jnp.sum(z, axis=0, keepdims=True)
  css = jnp.sum(z * z, axis=0, keepdims=True)

  @pl.when(k == 0)
  def _():
    s_ref[...] = cs
    ss_ref[...] = css

  @pl.when(k != 0)
  def _():
    s_ref[...] += cs
    ss_ref[...] += css


def _conv_mm1_body(h_ref, eps_ref, a0_ref, a1_ref, w_ref, b_ref,
                   z_ref, s_ref, ss_ref):
  k = pl.program_id(0)
  y = h_ref[...] * eps_ref[...] + a0_ref[...] + a1_ref[...]
  z = jnp.dot(y, w_ref[...], preferred_element_type=jnp.float32) + b_ref[...]
  z_ref[...] = z
  _stats_update(k, z, s_ref, ss_ref)


def _conv_mm1(h, eps, a0, a1, w, b):
  return pl.pallas_call(
      _conv_mm1_body,
      grid=(_NG,),
      in_specs=[
          pl.BlockSpec((_NRB, DF), lambda k: (k, 0)),
          pl.BlockSpec((1, DF), lambda k: (0, 0)),
          pl.BlockSpec((_NRB, DF), lambda k: (k, 0)),
          pl.BlockSpec((_NRB, DF), lambda k: (k, 0)),
          pl.BlockSpec((DF, HC), lambda k: (0, 0)),
          pl.BlockSpec((1, HC), lambda k: (0, 0)),
      ],
      out_specs=[
          pl.BlockSpec((_NRB, HC), lambda k: (k, 0)),
          pl.BlockSpec((1, HC), lambda k: (0, 0)),
          pl.BlockSpec((1, HC), lambda k: (0, 0)),
      ],
      out_shape=[
          jax.ShapeDtypeStruct((N, HC), jnp.float32),
          jax.ShapeDtypeStruct((1, HC), jnp.float32),
          jax.ShapeDtypeStruct((1, HC), jnp.float32),
      ],
  )(h, eps, a0, a1, w, b)


def _bn_cols(z, s, ss, g, c):
  m = s * (1.0 / N)
  v = ss * (1.0 / N) - m * m
  inv = g * lax.rsqrt(v + 1e-5)
  return (z - m) * inv + c


def _leaky(x):
  return jnp.where(x >= 0, x, 0.01 * x)


def _bn_mm2_body(z_ref, s_ref, ss_ref, g_ref, c_ref, w_ref, b_ref,
                 o_ref, s2_ref, ss2_ref):
  k = pl.program_id(0)
  t = _leaky(_bn_cols(z_ref[...], s_ref[...], ss_ref[...],
                      g_ref[...], c_ref[...]))
  z2 = jnp.dot(t, w_ref[...], preferred_element_type=jnp.float32) + b_ref[...]
  o_ref[...] = z2
  _stats_update(k, z2, s2_ref, ss2_ref)


def _bn_mm2(z, s, ss, g, c, w, b):
  dout = w.shape[1]
  return pl.pallas_call(
      _bn_mm2_body,
      grid=(_NG,),
      in_specs=[
          pl.BlockSpec((_NRB, HC), lambda k: (k, 0)),
          pl.BlockSpec((1, HC), lambda k: (0, 0)),
          pl.BlockSpec((1, HC), lambda k: (0, 0)),
          pl.BlockSpec((1, HC), lambda k: (0, 0)),
          pl.BlockSpec((1, HC), lambda k: (0, 0)),
          pl.BlockSpec((HC, dout), lambda k: (0, 0)),
          pl.BlockSpec((1, dout), lambda k: (0, 0)),
      ],
      out_specs=[
          pl.BlockSpec((_NRB, dout), lambda k: (k, 0)),
          pl.BlockSpec((1, dout), lambda k: (0, 0)),
          pl.BlockSpec((1, dout), lambda k: (0, 0)),
      ],
      out_shape=[
          jax.ShapeDtypeStruct((N, dout), jnp.float32),
          jax.ShapeDtypeStruct((1, dout), jnp.float32),
          jax.ShapeDtypeStruct((1, dout), jnp.float32),
      ],
  )(z, s, ss, g, c, w, b)


def _bn_leaky_body(z_ref, s_ref, ss_ref, g_ref, c_ref, o_ref):
  o_ref[...] = _leaky(_bn_cols(z_ref[...], s_ref[...], ss_ref[...],
                               g_ref[...], c_ref[...]))


def _bn_leaky(z, s, ss, g, c):
  return pl.pallas_call(
      _bn_leaky_body,
      grid=(_NG,),
      in_specs=[
          pl.BlockSpec((_NRB, HC), lambda k: (k, 0)),
          pl.BlockSpec((1, HC), lambda k: (0, 0)),
          pl.BlockSpec((1, HC), lambda k: (0, 0)),
          pl.BlockSpec((1, HC), lambda k: (0, 0)),
          pl.BlockSpec((1, HC), lambda k: (0, 0)),
      ],
      out_specs=pl.BlockSpec((_NRB, HC), lambda k: (k, 0)),
      out_shape=jax.ShapeDtypeStruct((N, HC), jnp.float32),
  )(z, s, ss, g, c)


def _final_mm1_body(h_ref, mx_ref, wa_ref, wb_ref, b_ref, z_ref, s_ref, ss_ref):
  k = pl.program_id(0)
  z = (jnp.dot(h_ref[...], wa_ref[...], preferred_element_type=jnp.float32)
       + jnp.dot(mx_ref[...], wb_ref[...], preferred_element_type=jnp.float32)
       + b_ref[...])
  z_ref[...] = z
  _stats_update(k, z, s_ref, ss_ref)


def _final_mm1(h, mx, wa, wb, b):
  return pl.pallas_call(
      _final_mm1_body,
      grid=(_NG,),
      in_specs=[
          pl.BlockSpec((_NRB, HC), lambda k: (k, 0)),
          pl.BlockSpec((_NRB, DM), lambda k: (k, 0)),
          pl.BlockSpec((HC, HF), lambda k: (0, 0)),
          pl.BlockSpec((DM, HF), lambda k: (0, 0)),
          pl.BlockSpec((1, HF), lambda k: (0, 0)),
      ],
      out_specs=[
          pl.BlockSpec((_NRB, HF), lambda k: (k, 0)),
          pl.BlockSpec((1, HF), lambda k: (0, 0)),
          pl.BlockSpec((1, HF), lambda k: (0, 0)),
      ],
      out_shape=[
          jax.ShapeDtypeStruct((N, HF), jnp.float32),
          jax.ShapeDtypeStruct((1, HF), jnp.float32),
          jax.ShapeDtypeStruct((1, HF), jnp.float32),
      ],
  )(h, mx, wa, wb, b)


def kernel(x, edge_index, edge_attr, mol_x, params):
  src = edge_index[0]
  dst = edge_index[1]

  def row(v):
    return v.reshape(1, -1)

  ea2 = edge_attr.reshape(E2, 2 * DE)
  es = []
  for i in range(3):
    p = params["conv%d" % i]
    w2 = jnp.zeros((2 * DE, 2 * DF), jnp.float32)
    w2 = w2.at[:DE, :DF].set(p["We"]).at[DE:, DF:].set(p["We"])
    b2 = jnp.concatenate([p["be"], p["be"]]).reshape(1, 2 * DF)
    es.append(_edge_mm(ea2, w2, b2))

  h = x
  for i in range(3):
    p = params["conv%d" % i]
    parts = _edge_agg(h, es[i], src, dst)
    epsb = jnp.broadcast_to(1.0 + p["eps"], (1, DF)).astype(jnp.float32)
    z1, s1, ss1 = _conv_mm1(h, epsb, parts[0], parts[1],
                            p["W1"], row(p["b1"]))
    z2, s2, ss2 = _bn_mm2(z1, s1, ss1, row(p["g1"]), row(p["c1"]),
                          p["W2"], row(p["b2"]))
    if i != 2:
      h = _bn_leaky(z2, s2, ss2, row(p["go"]), row(p["co"]))
    else:
      h = z2

  pf = params["final"]
  wa = pf["W1"][:HC]
  wb = pf["W1"][HC:]
  o1, fs, fss = _final_mm1(h, mol_x, wa, wb, row(pf["b1"]))
  w2p = jnp.zeros((HF, 128), jnp.float32).at[:, :1].set(pf["W2"])
  b2p = jnp.zeros((1, 128), jnp.float32).at[0, 0].set(pf["b2"][0])
  o, _, _ = _bn_mm2(o1, fs, fss, row(pf["g"]), row(pf["c"]), w2p, b2p)
  return o[:, 0]
